# baseline (speedup 1.0000x reference)
import jax
import jax.numpy as jnp
from jax import lax
from jax.experimental import pallas as pl
from jax.experimental.pallas import tpu as pltpu
from jax.experimental.pallas import tpu_sc as plsc

NUM_CLASSES = 100000
FEATURE_DIM = 64
BATCH = 16384

_NC, _NS, _L = 2, 16, 16
_NW = _NC * _NS
_DPW = FEATURE_DIM // _NW
_NQ = 4
_QB = BATCH // _NQ
_UNROLL = 4
_CHUNKS = _QB // (_L * _UNROLL)


def _center_loss_body(feats_t_hbm, labels_hbm, centers_t_hbm, out_hbm,
                      crow_v, lab_v, frow0_v, frow1_v, acc_v,
                      csem, lsem, fsem0, fsem1):
    wid = lax.axis_index("s") * _NC + lax.axis_index("c")
    frows = (frow0_v, frow1_v)
    fsems = (fsem0, fsem1)

    pltpu.make_async_copy(labels_hbm, lab_v, lsem).start()

    def make_quarter(dim, q):
        return pltpu.make_async_copy(
            feats_t_hbm.at[dim, pl.ds(q * _QB, _QB)], frows[q % 2], fsems[q % 2])

    def dim_body(j, accs):
        dim = wid + j * _NW
        crow_cp = pltpu.make_async_copy(centers_t_hbm.at[dim], crow_v, csem)
        crow_cp.start()
        make_quarter(dim, 0).start()

        @pl.when(j == 0)
        def _():
            pltpu.make_async_copy(labels_hbm, lab_v, lsem).wait()

        crow_cp.wait()
        for q in range(_NQ):
            make_quarter(dim, q).wait()
            if q + 1 < _NQ:
                make_quarter(dim, q + 1).start()
            frow = frows[q % 2]
            lab_base = q * _QB

            def chunk(k, accs, _lab_base=lab_base, _frow=frow):
                a0, a1 = accs
                base = k * (_L * _UNROLL)
                for u in range(_UNROLL):
                    off = base + u * _L
                    idx = lab_v[pl.ds(_lab_base + off, _L)]
                    c = plsc.load_gather(crow_v, [idx])
                    f = _frow[pl.ds(off, _L)]
                    d = f - c
                    if u % 2 == 0:
                        a0 = a0 + d * d
                    else:
                        a1 = a1 + d * d
                return a0, a1

            accs = plsc.parallel_loop(0, _CHUNKS, unroll=2, carry=accs)(chunk)
        return accs

    zero = jnp.zeros((_L,), jnp.float32)
    accs = lax.fori_loop(0, _DPW, dim_body, (zero, zero))

    acc_v[...] = accs[0] + accs[1]
    pltpu.sync_copy(acc_v, out_hbm.at[pl.ds(wid * _L, _L)])


@jax.jit
def _center_loss(features, labels, centers):
    mesh = plsc.VectorSubcoreMesh(core_axis_name="c", subcore_axis_name="s")
    partials = pl.kernel(
        _center_loss_body,
        out_type=jax.ShapeDtypeStruct((_NW * _L,), jnp.float32),
        mesh=mesh,
        compiler_params=pltpu.CompilerParams(needs_layout_passes=False, disable_bounds_checks=True),
        scratch_types=[
            pltpu.VMEM((NUM_CLASSES,), jnp.float32),
            pltpu.VMEM((BATCH,), jnp.int32),
            pltpu.VMEM((_QB,), jnp.float32),
            pltpu.VMEM((_QB,), jnp.float32),
            pltpu.VMEM((_L,), jnp.float32),
            pltpu.SemaphoreType.DMA,
            pltpu.SemaphoreType.DMA,
            pltpu.SemaphoreType.DMA,
            pltpu.SemaphoreType.DMA,
        ],
    )(features.T, labels.astype(jnp.int32), centers.T)
    return jnp.sum(partials) / (BATCH * FEATURE_DIM)


def kernel(features, labels, centers):
    return _center_loss(features, labels, centers)


# speedup vs baseline: 1.0095x; 1.0095x over previous
import jax
import jax.numpy as jnp
from jax import lax
from jax.experimental import pallas as pl
from jax.experimental.pallas import tpu as pltpu
from jax.experimental.pallas import tpu_sc as plsc

NUM_CLASSES = 100000
FEATURE_DIM = 64
BATCH = 16384

_NC, _NS, _L = 2, 16, 16
_NW = _NC * _NS
_DPW = FEATURE_DIM // _NW
_NQ = 4
_QB = BATCH // _NQ
_UNROLL = 4
_CHUNKS = _QB // (_L * _UNROLL)


def _center_loss_body(feats_t_hbm, labels_hbm, centers_t_hbm, out_hbm,
                      crow_v, lab_v, frow0_v, frow1_v, acc_v,
                      csem, lsem, fsem0, fsem1):
    wid = lax.axis_index("s") * _NC + lax.axis_index("c")
    frows = (frow0_v, frow1_v)
    fsems = (fsem0, fsem1)

    pltpu.make_async_copy(labels_hbm, lab_v, lsem).start()

    def make_quarter(dim, q):
        return pltpu.make_async_copy(
            feats_t_hbm.at[dim, pl.ds(q * _QB, _QB)], frows[q % 2], fsems[q % 2])

    def dim_body(j, accs):
        dim = wid * _DPW + j
        crow_cp = pltpu.make_async_copy(centers_t_hbm.at[dim], crow_v, csem)
        crow_cp.start()
        make_quarter(dim, 0).start()

        @pl.when(j == 0)
        def _():
            pltpu.make_async_copy(labels_hbm, lab_v, lsem).wait()

        crow_cp.wait()
        for q in range(_NQ):
            make_quarter(dim, q).wait()
            if q + 1 < _NQ:
                make_quarter(dim, q + 1).start()
            frow = frows[q % 2]
            lab_base = q * _QB

            def chunk(k, accs, _lab_base=lab_base, _frow=frow):
                a0, a1 = accs
                base = k * (_L * _UNROLL)
                for u in range(_UNROLL):
                    off = base + u * _L
                    idx = lab_v[pl.ds(_lab_base + off, _L)]
                    c = plsc.load_gather(crow_v, [idx])
                    f = _frow[pl.ds(off, _L)]
                    d = f - c
                    if u % 2 == 0:
                        a0 = a0 + d * d
                    else:
                        a1 = a1 + d * d
                return a0, a1

            accs = plsc.parallel_loop(0, _CHUNKS, unroll=2, carry=accs)(chunk)
        return accs

    zero = jnp.zeros((_L,), jnp.float32)
    accs = lax.fori_loop(0, _DPW, dim_body, (zero, zero))

    acc_v[...] = accs[0] + accs[1]
    pltpu.sync_copy(acc_v, out_hbm.at[pl.ds(wid * _L, _L)])


@jax.jit
def _center_loss(features, labels, centers):
    mesh = plsc.VectorSubcoreMesh(core_axis_name="c", subcore_axis_name="s")
    partials = pl.kernel(
        _center_loss_body,
        out_type=jax.ShapeDtypeStruct((_NW * _L,), jnp.float32),
        mesh=mesh,
        compiler_params=pltpu.CompilerParams(needs_layout_passes=False, disable_bounds_checks=True),
        scratch_types=[
            pltpu.VMEM((NUM_CLASSES,), jnp.float32),
            pltpu.VMEM((BATCH,), jnp.int32),
            pltpu.VMEM((_QB,), jnp.float32),
            pltpu.VMEM((_QB,), jnp.float32),
            pltpu.VMEM((_L,), jnp.float32),
            pltpu.SemaphoreType.DMA,
            pltpu.SemaphoreType.DMA,
            pltpu.SemaphoreType.DMA,
            pltpu.SemaphoreType.DMA,
        ],
    )(features.T, labels.astype(jnp.int32), centers.T)
    return jnp.sum(partials) / (BATCH * FEATURE_DIM)


def kernel(features, labels, centers):
    return _center_loss(features, labels, centers)
